# baseline (speedup 1.0000x reference)
import functools

import jax
import jax.numpy as jnp
from jax import lax
from jax.experimental import pallas as pl
from jax.experimental.pallas import tpu as pltpu, tpu_sc as plsc

N = 10000
D = 128
H = 128
E = 320000
NC = 2
NS = 16
NW = NC * NS
CE = 1600
NCH = E // CE
ECNT = E // NW
EROWS = E // 128


def _fold_bn(g, b, m, v):
    s = g / jnp.sqrt(v + 1e-3)
    return s, b - m * s


def _gelu(z):
    return 0.5 * z * (1.0 + lax.erf(z * 0.7071067811865476))


def _prepare_body(x_ref, src_ref, dst_ref, w1_ref, w2_ref,
                  g1_ref, b1_ref, m1_ref, v1_ref, g2_ref, b2_ref, m2_ref,
                  v2_ref, c1_ref, c2_ref, ht_ref, sd_ref):
    s1, t1 = _fold_bn(g1_ref[...], b1_ref[...], m1_ref[...], v1_ref[...])
    s2, t2 = _fold_bn(g2_ref[...], b2_ref[...], m2_ref[...], v2_ref[...])
    xb = x_ref[...] * s1 + t1
    h1 = _gelu(jnp.dot(xb, w1_ref[...], preferred_element_type=jnp.float32)
               + c1_ref[...])
    hb = h1 * s2 + t2
    h = _gelu(jnp.dot(hb, w2_ref[...], preferred_element_type=jnp.float32)
              + c2_ref[...])
    au = lax.bitcast_convert_type(h[:, :H // 2].astype(jnp.bfloat16),
                                  jnp.uint16).astype(jnp.int32)
    bu = lax.bitcast_convert_type(h[:, H // 2:].astype(jnp.bfloat16),
                                  jnp.uint16).astype(jnp.int32)
    ht_ref[...] = (au | (bu << 16)).T
    sd_ref[...] = src_ref[...] | (dst_ref[...] << 16)


def _prepare_ffn(x, src2d, dst2d, w1, w2, bns, c1, c2):
    return pl.pallas_call(
        _prepare_body,
        out_shape=[jax.ShapeDtypeStruct((H // 2, N), jnp.int32),
                   jax.ShapeDtypeStruct((EROWS, 128), jnp.int32)],
    )(x, src2d, dst2d, w1, w2, *bns, c1, c2)


def _edge_body(ht_hbm, sd_hbm, wgt_hbm, dst_hbm, sums_hbm, cnt_hbm,
               htp0, htp1, acc0, acc1, acc2, acc3,
               sdb0, sdb1, wb0, wb1, cnt_buf, cdst_buf, sem0, sem1, semc, semh):
    wid = lax.axis_index("s") * NC + lax.axis_index("c")
    p0 = 2 * wid

    coff = pl.multiple_of(wid * ECNT, 8)
    pltpu.async_copy(dst_hbm.at[pl.ds(coff, ECNT)], cdst_buf, semc)

    pltpu.async_copy(ht_hbm.at[pl.ds(pl.multiple_of(p0 * N, 8), N)], htp0,
                     semh)
    pltpu.async_copy(ht_hbm.at[pl.ds(pl.multiple_of((p0 + 1) * N, 8), N)],
                     htp1, semh)
    pltpu.async_copy(sd_hbm.at[pl.ds(0, CE)], sdb0, sem0)
    pltpu.async_copy(wgt_hbm.at[pl.ds(0, CE)], wb0, sem0)
    pltpu.async_copy(sd_hbm.at[pl.ds(CE, CE)], sdb1, sem1)
    pltpu.async_copy(wgt_hbm.at[pl.ds(CE, CE)], wb1, sem1)

    for a in (acc0, acc1, acc2, acc3):
        @plsc.parallel_loop(0, N, step=16, unroll=8)
        def _zero_acc(j, a=a):
            a[pl.ds(j, 16)] = jnp.zeros((16,), jnp.float32)

    pltpu.make_async_copy(ht_hbm.at[pl.ds(0, N)], htp0, semh).wait()
    pltpu.make_async_copy(ht_hbm.at[pl.ds(0, N)], htp1, semh).wait()

    hi_mask = jnp.full((16,), -65536, jnp.int32)
    lo_mask = jnp.full((16,), 65535, jnp.int32)

    def _pair(g, _):
        for b, (sdb, wb, sem) in enumerate(((sdb0, wb0, sem0),
                                            (sdb1, wb1, sem1))):
            k = 2 * g + b
            pltpu.make_async_copy(sd_hbm.at[pl.ds(0, CE)], sdb, sem).wait()
            pltpu.make_async_copy(wgt_hbm.at[pl.ds(0, CE)], wb, sem).wait()

            @plsc.parallel_loop(0, CE, step=16, unroll=8)
            def _win(i):
                sd = sdb[pl.ds(i, 16)]
                wt = wb[pl.ds(i, 16)]
                s = sd & lo_mask
                d = lax.shift_right_logical(sd, 16)
                for htp, accl, acch in ((htp0, acc0, acc1),
                                        (htp1, acc2, acc3)):
                    g2 = plsc.load_gather(htp, [s])
                    lo = plsc.bitcast(lax.shift_left(g2, 16), jnp.float32)
                    hi = plsc.bitcast(g2 & hi_mask, jnp.float32)
                    plsc.addupdate_scatter(accl, [d], lo * wt)
                    plsc.addupdate_scatter(acch, [d], hi * wt)

            off = pl.multiple_of(lax.rem((k + 2) * CE, E), 8)
            pltpu.async_copy(sd_hbm.at[pl.ds(off, CE)], sdb, sem)
            pltpu.async_copy(wgt_hbm.at[pl.ds(off, CE)], wb, sem)
        return 0
    lax.fori_loop(0, NCH // 2, _pair, 0)
    pltpu.make_async_copy(sd_hbm.at[pl.ds(0, CE)], sdb0, sem0).wait()
    pltpu.make_async_copy(wgt_hbm.at[pl.ds(0, CE)], wb0, sem0).wait()
    pltpu.make_async_copy(sd_hbm.at[pl.ds(0, CE)], sdb1, sem1).wait()
    pltpu.make_async_copy(wgt_hbm.at[pl.ds(0, CE)], wb1, sem1).wait()

    for a, frow in ((acc0, p0), (acc1, p0 + H // 2),
                    (acc2, p0 + 1), (acc3, p0 + 1 + H // 2)):
        pltpu.async_copy(a, sums_hbm.at[pl.ds(pl.multiple_of(frow * N, 8), N)],
                         semh)

    pltpu.make_async_copy(dst_hbm.at[pl.ds(0, ECNT)], cdst_buf, semc).wait()

    @plsc.parallel_loop(0, N, step=16, unroll=8)
    def _zero_cnt(j):
        cnt_buf[pl.ds(j, 16)] = jnp.zeros((16,), jnp.float32)

    ones = jnp.full((16,), 1.0, jnp.float32)

    @plsc.parallel_loop(0, ECNT, step=16, unroll=8)
    def _cwin(i):
        d = cdst_buf[pl.ds(i, 16)]
        plsc.addupdate_scatter(cnt_buf, [d], ones)

    for a in (acc0, acc1, acc2, acc3):
        pltpu.make_async_copy(a, sums_hbm.at[pl.ds(0, N)], semh).wait()
    pltpu.sync_copy(cnt_buf, cnt_hbm.at[pl.ds(pl.multiple_of(wid * N, 8), N)])


@functools.cache
def _edge_kernel():
    return pl.kernel(
        _edge_body,
        out_type=[jax.ShapeDtypeStruct((D * N,), jnp.float32),
                  jax.ShapeDtypeStruct((NW * N,), jnp.float32)],
        mesh=plsc.VectorSubcoreMesh(core_axis_name="c", subcore_axis_name="s",
                                    num_cores=NC, num_subcores=NS),
        compiler_params=pltpu.CompilerParams(needs_layout_passes=False),
        scratch_types=[pltpu.VMEM((N,), jnp.int32),
                       pltpu.VMEM((N,), jnp.int32),
                       pltpu.VMEM((N,), jnp.float32),
                       pltpu.VMEM((N,), jnp.float32),
                       pltpu.VMEM((N,), jnp.float32),
                       pltpu.VMEM((N,), jnp.float32),
                       pltpu.VMEM((CE,), jnp.int32),
                       pltpu.VMEM((CE,), jnp.int32),
                       pltpu.VMEM((CE,), jnp.float32),
                       pltpu.VMEM((CE,), jnp.float32),
                       pltpu.VMEM((N,), jnp.float32),
                       pltpu.VMEM((ECNT,), jnp.int32),
                       pltpu.SemaphoreType.DMA,
                       pltpu.SemaphoreType.DMA,
                       pltpu.SemaphoreType.DMA,
                       pltpu.SemaphoreType.DMA])


def _update_body(x_ref, sums_ref, cntp_ref, uw1_ref, uw2_ref,
                 ug1_ref, ub1_ref, um1_ref, uv1_ref, ug2_ref, ub2_ref,
                 um2_ref, uv2_ref, uc1_ref, uc2_ref, out_ref):
    s1, t1 = _fold_bn(ug1_ref[...], ub1_ref[...], um1_ref[...], uv1_ref[...])
    s2, t2 = _fold_bn(ug2_ref[...], ub2_ref[...], um2_ref[...], uv2_ref[...])
    cnt = jnp.sum(cntp_ref[...], axis=0, keepdims=True)
    agg = (sums_ref[...] / jnp.maximum(cnt, 1.0)).T
    xb = x_ref[...] * s1[:, :D] + t1[:, :D]
    ab = agg * s1[:, D:] + t1[:, D:]
    z1 = (jnp.dot(xb, uw1_ref[:D], preferred_element_type=jnp.float32)
          + jnp.dot(ab, uw1_ref[D:], preferred_element_type=jnp.float32)
          + uc1_ref[...])
    h1 = _gelu(z1)
    hb = h1 * s2 + t2
    out_ref[...] = _gelu(jnp.dot(hb, uw2_ref[...],
                                 preferred_element_type=jnp.float32)
                         + uc2_ref[...])


def _update_ffn(x, sums_t, cntp, uw1, uw2, ubns, uc1, uc2):
    return pl.pallas_call(
        _update_body,
        out_shape=jax.ShapeDtypeStruct((N, H), jnp.float32),
    )(x, sums_t, cntp, uw1, uw2, *ubns, uc1, uc2)


def kernel(x, edges, edge_weights, g1, b1, m1, v1, W1, c1, g2, b2, m2, v2,
           W2, c2, ug1, ub1, um1, uv1, UW1, uc1, ug2, ub2, um2, uv2, UW2, uc2):
    def rowv(p):
        return p.reshape(1, -1)

    src2d = edges[1].reshape(EROWS, 128)
    dst2d = edges[0].reshape(EROWS, 128)

    ht, sd = _prepare_ffn(
        x, src2d, dst2d, W1, W2,
        [rowv(p) for p in (g1, b1, m1, v1, g2, b2, m2, v2)],
        rowv(c1), rowv(c2))

    sums_flat, cnt_flat = _edge_kernel()(
        ht.reshape(-1), sd.reshape(-1), edge_weights, edges[0])

    return _update_ffn(
        x, sums_flat.reshape(D, N), cnt_flat.reshape(NW, N), UW1, UW2,
        [rowv(p) for p in (ug1, ub1, um1, uv1, ug2, ub2, um2, uv2)],
        rowv(uc1), rowv(uc2))


# speedup vs baseline: 1.0480x; 1.0480x over previous
import functools

import jax
import jax.numpy as jnp
from jax import lax
from jax.experimental import pallas as pl
from jax.experimental.pallas import tpu as pltpu, tpu_sc as plsc

N = 10000
D = 128
H = 128
E = 320000
NC = 2
NS = 16
NW = NC * NS
CE = 3200
NCH = E // CE
ECNT = E // NW
EROWS = E // 128


def _fold_bn(g, b, m, v):
    s = g / jnp.sqrt(v + 1e-3)
    return s, b - m * s


def _gelu(z):
    return 0.5 * z * (1.0 + lax.erf(z * 0.7071067811865476))


def _prepare_body(x_ref, src_ref, dst_ref, w1_ref, w2_ref,
                  g1_ref, b1_ref, m1_ref, v1_ref, g2_ref, b2_ref, m2_ref,
                  v2_ref, c1_ref, c2_ref, ht_ref, sd_ref):
    s1, t1 = _fold_bn(g1_ref[...], b1_ref[...], m1_ref[...], v1_ref[...])
    s2, t2 = _fold_bn(g2_ref[...], b2_ref[...], m2_ref[...], v2_ref[...])
    xb = x_ref[...] * s1 + t1
    h1 = _gelu(jnp.dot(xb, w1_ref[...], preferred_element_type=jnp.float32)
               + c1_ref[...])
    hb = h1 * s2 + t2
    h = _gelu(jnp.dot(hb, w2_ref[...], preferred_element_type=jnp.float32)
              + c2_ref[...])
    au = lax.bitcast_convert_type(h[:, :H // 2].astype(jnp.bfloat16),
                                  jnp.uint16).astype(jnp.int32)
    bu = lax.bitcast_convert_type(h[:, H // 2:].astype(jnp.bfloat16),
                                  jnp.uint16).astype(jnp.int32)
    ht_ref[...] = (au | (bu << 16)).T
    sd_ref[...] = src_ref[...] | (dst_ref[...] << 16)


def _prepare_ffn(x, src2d, dst2d, w1, w2, bns, c1, c2):
    return pl.pallas_call(
        _prepare_body,
        out_shape=[jax.ShapeDtypeStruct((H // 2, N), jnp.int32),
                   jax.ShapeDtypeStruct((EROWS, 128), jnp.int32)],
    )(x, src2d, dst2d, w1, w2, *bns, c1, c2)


def _edge_body(ht_hbm, sd_hbm, wgt_hbm, dst_hbm, sums_hbm, cnt_hbm,
               htp0, htp1, acc0, acc1, acc2, acc3,
               sdb0, sdb1, wb0, wb1, cnt_buf, cdst_buf, sem0, sem1, semc, semh):
    wid = lax.axis_index("s") * NC + lax.axis_index("c")
    p0 = 2 * wid

    coff = pl.multiple_of(wid * ECNT, 8)
    pltpu.async_copy(dst_hbm.at[pl.ds(coff, ECNT)], cdst_buf, semc)

    pltpu.async_copy(ht_hbm.at[pl.ds(pl.multiple_of(p0 * N, 8), N)], htp0,
                     semh)
    pltpu.async_copy(ht_hbm.at[pl.ds(pl.multiple_of((p0 + 1) * N, 8), N)],
                     htp1, semh)
    pltpu.async_copy(sd_hbm.at[pl.ds(0, CE)], sdb0, sem0)
    pltpu.async_copy(wgt_hbm.at[pl.ds(0, CE)], wb0, sem0)
    pltpu.async_copy(sd_hbm.at[pl.ds(CE, CE)], sdb1, sem1)
    pltpu.async_copy(wgt_hbm.at[pl.ds(CE, CE)], wb1, sem1)

    for a in (acc0, acc1, acc2, acc3):
        @plsc.parallel_loop(0, N, step=16, unroll=8)
        def _zero_acc(j, a=a):
            a[pl.ds(j, 16)] = jnp.zeros((16,), jnp.float32)

    pltpu.make_async_copy(ht_hbm.at[pl.ds(0, N)], htp0, semh).wait()
    pltpu.make_async_copy(ht_hbm.at[pl.ds(0, N)], htp1, semh).wait()

    hi_mask = jnp.full((16,), -65536, jnp.int32)
    lo_mask = jnp.full((16,), 65535, jnp.int32)

    def _pair(g, _):
        for b, (sdb, wb, sem) in enumerate(((sdb0, wb0, sem0),
                                            (sdb1, wb1, sem1))):
            k = 2 * g + b
            pltpu.make_async_copy(sd_hbm.at[pl.ds(0, CE)], sdb, sem).wait()
            pltpu.make_async_copy(wgt_hbm.at[pl.ds(0, CE)], wb, sem).wait()

            @plsc.parallel_loop(0, CE, step=16, unroll=8)
            def _win(i):
                sd = sdb[pl.ds(i, 16)]
                wt = wb[pl.ds(i, 16)]
                s = sd & lo_mask
                d = lax.shift_right_logical(sd, 16)
                for htp, accl, acch in ((htp0, acc0, acc1),
                                        (htp1, acc2, acc3)):
                    g2 = plsc.load_gather(htp, [s])
                    lo = plsc.bitcast(lax.shift_left(g2, 16), jnp.float32)
                    hi = plsc.bitcast(g2 & hi_mask, jnp.float32)
                    plsc.addupdate_scatter(accl, [d], lo * wt)
                    plsc.addupdate_scatter(acch, [d], hi * wt)

            off = pl.multiple_of(lax.rem((k + 2) * CE, E), 8)
            pltpu.async_copy(sd_hbm.at[pl.ds(off, CE)], sdb, sem)
            pltpu.async_copy(wgt_hbm.at[pl.ds(off, CE)], wb, sem)
        return 0
    lax.fori_loop(0, NCH // 2, _pair, 0)
    pltpu.make_async_copy(sd_hbm.at[pl.ds(0, CE)], sdb0, sem0).wait()
    pltpu.make_async_copy(wgt_hbm.at[pl.ds(0, CE)], wb0, sem0).wait()
    pltpu.make_async_copy(sd_hbm.at[pl.ds(0, CE)], sdb1, sem1).wait()
    pltpu.make_async_copy(wgt_hbm.at[pl.ds(0, CE)], wb1, sem1).wait()

    for a, frow in ((acc0, p0), (acc1, p0 + H // 2),
                    (acc2, p0 + 1), (acc3, p0 + 1 + H // 2)):
        pltpu.async_copy(a, sums_hbm.at[pl.ds(pl.multiple_of(frow * N, 8), N)],
                         semh)

    pltpu.make_async_copy(dst_hbm.at[pl.ds(0, ECNT)], cdst_buf, semc).wait()

    @plsc.parallel_loop(0, N, step=16, unroll=8)
    def _zero_cnt(j):
        cnt_buf[pl.ds(j, 16)] = jnp.zeros((16,), jnp.float32)

    ones = jnp.full((16,), 1.0, jnp.float32)

    @plsc.parallel_loop(0, ECNT, step=16, unroll=8)
    def _cwin(i):
        d = cdst_buf[pl.ds(i, 16)]
        plsc.addupdate_scatter(cnt_buf, [d], ones)

    for a in (acc0, acc1, acc2, acc3):
        pltpu.make_async_copy(a, sums_hbm.at[pl.ds(0, N)], semh).wait()
    pltpu.sync_copy(cnt_buf, cnt_hbm.at[pl.ds(pl.multiple_of(wid * N, 8), N)])


@functools.cache
def _edge_kernel():
    return pl.kernel(
        _edge_body,
        out_type=[jax.ShapeDtypeStruct((D * N,), jnp.float32),
                  jax.ShapeDtypeStruct((NW * N,), jnp.float32)],
        mesh=plsc.VectorSubcoreMesh(core_axis_name="c", subcore_axis_name="s",
                                    num_cores=NC, num_subcores=NS),
        compiler_params=pltpu.CompilerParams(needs_layout_passes=False),
        scratch_types=[pltpu.VMEM((N,), jnp.int32),
                       pltpu.VMEM((N,), jnp.int32),
                       pltpu.VMEM((N,), jnp.float32),
                       pltpu.VMEM((N,), jnp.float32),
                       pltpu.VMEM((N,), jnp.float32),
                       pltpu.VMEM((N,), jnp.float32),
                       pltpu.VMEM((CE,), jnp.int32),
                       pltpu.VMEM((CE,), jnp.int32),
                       pltpu.VMEM((CE,), jnp.float32),
                       pltpu.VMEM((CE,), jnp.float32),
                       pltpu.VMEM((N,), jnp.float32),
                       pltpu.VMEM((ECNT,), jnp.int32),
                       pltpu.SemaphoreType.DMA,
                       pltpu.SemaphoreType.DMA,
                       pltpu.SemaphoreType.DMA,
                       pltpu.SemaphoreType.DMA])


def _update_body(x_ref, sums_ref, cntp_ref, uw1_ref, uw2_ref,
                 ug1_ref, ub1_ref, um1_ref, uv1_ref, ug2_ref, ub2_ref,
                 um2_ref, uv2_ref, uc1_ref, uc2_ref, out_ref):
    s1, t1 = _fold_bn(ug1_ref[...], ub1_ref[...], um1_ref[...], uv1_ref[...])
    s2, t2 = _fold_bn(ug2_ref[...], ub2_ref[...], um2_ref[...], uv2_ref[...])
    cnt = jnp.sum(cntp_ref[...], axis=0, keepdims=True)
    agg = (sums_ref[...] / jnp.maximum(cnt, 1.0)).T
    xb = x_ref[...] * s1[:, :D] + t1[:, :D]
    ab = agg * s1[:, D:] + t1[:, D:]
    z1 = (jnp.dot(xb, uw1_ref[:D], preferred_element_type=jnp.float32)
          + jnp.dot(ab, uw1_ref[D:], preferred_element_type=jnp.float32)
          + uc1_ref[...])
    h1 = _gelu(z1)
    hb = h1 * s2 + t2
    out_ref[...] = _gelu(jnp.dot(hb, uw2_ref[...],
                                 preferred_element_type=jnp.float32)
                         + uc2_ref[...])


def _update_ffn(x, sums_t, cntp, uw1, uw2, ubns, uc1, uc2):
    return pl.pallas_call(
        _update_body,
        out_shape=jax.ShapeDtypeStruct((N, H), jnp.float32),
    )(x, sums_t, cntp, uw1, uw2, *ubns, uc1, uc2)


def kernel(x, edges, edge_weights, g1, b1, m1, v1, W1, c1, g2, b2, m2, v2,
           W2, c2, ug1, ub1, um1, uv1, UW1, uc1, ug2, ub2, um2, uv2, UW2, uc2):
    def rowv(p):
        return p.reshape(1, -1)

    src2d = edges[1].reshape(EROWS, 128)
    dst2d = edges[0].reshape(EROWS, 128)

    ht, sd = _prepare_ffn(
        x, src2d, dst2d, W1, W2,
        [rowv(p) for p in (g1, b1, m1, v1, g2, b2, m2, v2)],
        rowv(c1), rowv(c2))

    sums_flat, cnt_flat = _edge_kernel()(
        ht.reshape(-1), sd.reshape(-1), edge_weights, edges[0])

    return _update_ffn(
        x, sums_flat.reshape(D, N), cnt_flat.reshape(NW, N), UW1, UW2,
        [rowv(p) for p in (ug1, ub1, um1, uv1, ug2, ub2, um2, uv2)],
        rowv(uc1), rowv(uc2))
